# baseline (speedup 1.0000x reference)
import jax
import jax.numpy as jnp
from jax import lax
from jax.experimental import pallas as pl
from jax.experimental.pallas import tpu as pltpu
from jax.experimental.pallas import tpu_sc as plsc

B = 4096
L = 200
D = 128
N_IDX = B * L

NUM_CORES = 2
NUM_SUBCORES = 16
NW = NUM_CORES * NUM_SUBCORES
ROWS_PER_W = N_IDX // NW

NB = 5
LOOKAHEAD = 3
CHUNK = 128
N_CHUNKS = ROWS_PER_W // CHUNK
N_STEPS = N_CHUNKS // NB
IDX_ROWS_PER_W = ROWS_PER_W // CHUNK


def _embed_body(x_hbm, table_hbm, out_hbm, idx_v, *scratch):
    rows = scratch[:NB]
    gsem = scratch[NB:2 * NB]
    osem = scratch[2 * NB:3 * NB]

    wid = lax.axis_index("s") * NUM_CORES + lax.axis_index("c")
    out_base = wid * ROWS_PER_W

    pltpu.sync_copy(x_hbm.at[pl.ds(wid * IDX_ROWS_PER_W, IDX_ROWS_PER_W)], idx_v)

    def fire_gather(g, b):
        pltpu.async_copy(table_hbm.at[idx_v.at[g]], rows[b], gsem[b])

    for g in range(LOOKAHEAD):
        fire_gather(g, g % NB)

    def step(i, _):
        for b in range(NB):
            g = i * NB + b
            ba = (b + LOOKAHEAD) % NB

            @pl.when((g >= NB - LOOKAHEAD) & (g < N_CHUNKS - LOOKAHEAD))
            def _wait_out():
                pltpu.make_async_copy(
                    rows[ba], out_hbm.at[pl.ds(out_base, CHUNK)], osem[ba]
                ).wait()

            @pl.when(g < N_CHUNKS - LOOKAHEAD)
            def _fire_ahead():
                fire_gather(g + LOOKAHEAD, ba)

            pltpu.make_async_copy(
                table_hbm.at[idx_v.at[g]], rows[b], gsem[b]
            ).wait()
            pltpu.async_copy(
                rows[b], out_hbm.at[pl.ds(out_base + g * CHUNK, CHUNK)], osem[b]
            )
        return 0

    lax.fori_loop(0, N_STEPS, step, 0)

    for b in range(NB):
        pltpu.make_async_copy(
            rows[b], out_hbm.at[pl.ds(out_base, CHUNK)], osem[b]
        ).wait()


@jax.jit
def _embed(x2d, table):
    mesh = plsc.VectorSubcoreMesh(core_axis_name="c", subcore_axis_name="s")
    return pl.kernel(
        _embed_body,
        mesh=mesh,
        out_type=jax.ShapeDtypeStruct((N_IDX, D), jnp.float32),
        scratch_types=(
            [pltpu.VMEM((IDX_ROWS_PER_W, CHUNK), jnp.int32)]
            + [pltpu.VMEM((CHUNK, D), jnp.float32) for _ in range(NB)]
            + [pltpu.SemaphoreType.DMA for _ in range(2 * NB)]
        ),
    )(x2d, table)


def kernel(x, table):
    x2d = x.reshape(N_IDX // CHUNK, CHUNK).astype(jnp.int32)
    out = _embed(x2d, table)
    return out.reshape(B, L, D)


# speedup vs baseline: 1.0428x; 1.0428x over previous
import jax
import jax.numpy as jnp
from jax import lax
from jax.experimental import pallas as pl
from jax.experimental.pallas import tpu as pltpu
from jax.experimental.pallas import tpu_sc as plsc

B = 4096
L = 200
D = 128
N_IDX = B * L

NUM_CORES = 2
NUM_SUBCORES = 16
NW = NUM_CORES * NUM_SUBCORES
ROWS_PER_W = N_IDX // NW

NB = 4
NBS = 2
LOOKAHEAD = 3
CHUNK = 128
N_CHUNKS = ROWS_PER_W // CHUNK
N_STEPS = N_CHUNKS // NB
IDX_ROWS_PER_W = ROWS_PER_W // CHUNK


def _embed_body(x_hbm, table_hbm, out_hbm, idx_v, shared, *scratch):
    rows = scratch[:NB]
    gsem = scratch[NB:2 * NB]
    csem = scratch[2 * NB:2 * NB + NBS]
    osem = scratch[2 * NB + NBS:2 * NB + 2 * NBS]

    cid = lax.axis_index("c")
    sid = lax.axis_index("s")
    wid = sid * NUM_CORES + cid
    out_base = wid * ROWS_PER_W

    pltpu.sync_copy(x_hbm.at[pl.ds(wid * IDX_ROWS_PER_W, IDX_ROWS_PER_W)], idx_v)

    def fire_gather(g, b):
        pltpu.async_copy(table_hbm.at[idx_v.at[g]], rows[b], gsem[b])

    for g in range(LOOKAHEAD):
        fire_gather(g, g % NB)

    def step(i, _):
        for b in range(NB):
            g = i * NB + b
            ba = (b + LOOKAHEAD) % NB
            bs = b % NBS
            bp = (b - 1) % NBS

            @pl.when(g >= NBS)
            def _wait_out():
                pltpu.make_async_copy(
                    shared.at[sid, bs], out_hbm.at[pl.ds(out_base, CHUNK)], osem[bs]
                ).wait()

            pltpu.make_async_copy(
                table_hbm.at[idx_v.at[g]], rows[b], gsem[b]
            ).wait()
            pltpu.async_copy(rows[b], shared.at[sid, bs], csem[bs])

            @pl.when(g >= 1)
            def _out_prev():
                pltpu.make_async_copy(
                    rows[b], shared.at[sid, bp], csem[bp]
                ).wait()
                pltpu.async_copy(
                    shared.at[sid, bp],
                    out_hbm.at[pl.ds(out_base + (g - 1) * CHUNK, CHUNK)],
                    osem[bp],
                )

            @pl.when(g < N_CHUNKS - LOOKAHEAD)
            def _fire_ahead():
                fire_gather(g + LOOKAHEAD, ba)
        return 0

    lax.fori_loop(0, N_STEPS, step, 0)

    last = N_CHUNKS - 1
    bl = last % NBS
    pltpu.make_async_copy(rows[last % NB], shared.at[sid, bl], csem[bl]).wait()
    pltpu.async_copy(
        shared.at[sid, bl], out_hbm.at[pl.ds(out_base + last * CHUNK, CHUNK)], osem[bl]
    )
    for b in range(NBS):
        pltpu.make_async_copy(
            shared.at[sid, b], out_hbm.at[pl.ds(out_base, CHUNK)], osem[b]
        ).wait()


@jax.jit
def _embed(x2d, table):
    mesh = plsc.VectorSubcoreMesh(core_axis_name="c", subcore_axis_name="s")
    return pl.kernel(
        _embed_body,
        mesh=mesh,
        out_type=jax.ShapeDtypeStruct((N_IDX, D), jnp.float32),
        scratch_types=(
            [pltpu.VMEM((IDX_ROWS_PER_W, CHUNK), jnp.int32)]
            + [pltpu.VMEM_SHARED((NUM_SUBCORES, NBS, CHUNK, D), jnp.float32)]
            + [pltpu.VMEM((CHUNK, D), jnp.float32) for _ in range(NB)]
            + [pltpu.SemaphoreType.DMA for _ in range(2 * NB + 2 * NBS)]
        ),
    )(x2d, table)


def kernel(x, table):
    x2d = x.reshape(N_IDX // CHUNK, CHUNK).astype(jnp.int32)
    out = _embed(x2d, table)
    return out.reshape(B, L, D)


# speedup vs baseline: 1.0487x; 1.0056x over previous
import jax
import jax.numpy as jnp
from jax import lax
from jax.experimental import pallas as pl
from jax.experimental.pallas import tpu as pltpu
from jax.experimental.pallas import tpu_sc as plsc

B = 4096
L = 200
D = 128
N_IDX = B * L

NUM_CORES = 2
NUM_SUBCORES = 16
NW = NUM_CORES * NUM_SUBCORES
ROWS_PER_W = N_IDX // NW

NB = 4
NBS = 2
LOOKAHEAD = 3
CHUNK = 128
N_CHUNKS = ROWS_PER_W // CHUNK
N_STEPS = N_CHUNKS // NB
IDX_ROWS_PER_W = ROWS_PER_W // CHUNK


def _embed_body(x_hbm, table_hbm, out_hbm, idx_v, shared, *scratch):
    rows = scratch[:NB]
    gsem = scratch[NB:2 * NB]
    csem = scratch[2 * NB:2 * NB + NBS]
    osem = scratch[2 * NB + NBS:2 * NB + 2 * NBS]

    cid = lax.axis_index("c")
    sid = lax.axis_index("s")
    wid = sid * NUM_CORES + cid
    out_base = wid * ROWS_PER_W

    pltpu.sync_copy(x_hbm.at[pl.ds(wid * IDX_ROWS_PER_W, IDX_ROWS_PER_W)], idx_v)

    def fire_gather(g, b):
        pltpu.async_copy(table_hbm.at[idx_v.at[g]], rows[b], gsem[b])

    for g in range(LOOKAHEAD):
        fire_gather(g, g % NB)

    def step(i, _):
        for b in range(NB):
            g = i * NB + b
            ba = (b + LOOKAHEAD) % NB
            bs = b % NBS
            bp = (b - 1) % NBS

            @pl.when(g >= 1)
            def _out_prev():
                pltpu.make_async_copy(
                    rows[b], shared.at[sid, bp], csem[bp]
                ).wait()
                pltpu.async_copy(
                    shared.at[sid, bp],
                    out_hbm.at[pl.ds(out_base + (g - 1) * CHUNK, CHUNK)],
                    osem[bp],
                )

            @pl.when(g < N_CHUNKS - LOOKAHEAD)
            def _fire_ahead():
                fire_gather(g + LOOKAHEAD, ba)

            @pl.when(g >= NBS)
            def _wait_out():
                pltpu.make_async_copy(
                    shared.at[sid, bs], out_hbm.at[pl.ds(out_base, CHUNK)], osem[bs]
                ).wait()

            pltpu.make_async_copy(
                table_hbm.at[idx_v.at[g]], rows[b], gsem[b]
            ).wait()
            pltpu.async_copy(rows[b], shared.at[sid, bs], csem[bs])
        return 0

    lax.fori_loop(0, N_STEPS, step, 0)

    last = N_CHUNKS - 1
    bl = last % NBS
    pltpu.make_async_copy(rows[last % NB], shared.at[sid, bl], csem[bl]).wait()
    pltpu.async_copy(
        shared.at[sid, bl], out_hbm.at[pl.ds(out_base + last * CHUNK, CHUNK)], osem[bl]
    )
    for b in range(NBS):
        pltpu.make_async_copy(
            shared.at[sid, b], out_hbm.at[pl.ds(out_base, CHUNK)], osem[b]
        ).wait()


@jax.jit
def _embed(x2d, table):
    mesh = plsc.VectorSubcoreMesh(core_axis_name="c", subcore_axis_name="s")
    return pl.kernel(
        _embed_body,
        mesh=mesh,
        out_type=jax.ShapeDtypeStruct((N_IDX, D), jnp.float32),
        scratch_types=(
            [pltpu.VMEM((IDX_ROWS_PER_W, CHUNK), jnp.int32)]
            + [pltpu.VMEM_SHARED((NUM_SUBCORES, NBS, CHUNK, D), jnp.float32)]
            + [pltpu.VMEM((CHUNK, D), jnp.float32) for _ in range(NB)]
            + [pltpu.SemaphoreType.DMA for _ in range(2 * NB + 2 * NBS)]
        ),
    )(x2d, table)


def kernel(x, table):
    x2d = x.reshape(N_IDX // CHUNK, CHUNK).astype(jnp.int32)
    out = _embed(x2d, table)
    return out.reshape(B, L, D)
